# Initial kernel scaffold; baseline (speedup 1.0000x reference)
import jax
import jax.numpy as jnp
from jax.experimental import pallas as pl


def kernel(state, action, weights):
    raise NotImplementedError("write your pallas kernel here")


# speedup vs baseline: 4.0898x; 4.0898x over previous
import functools

import jax
import jax.numpy as jnp
from jax import lax
from jax.experimental import pallas as pl
from jax.experimental.pallas import tpu as pltpu
from jax.experimental.pallas import tpu_sc as plsc

_NC = 2
_NS = 16
_NW = _NC * _NS
_CB = 64
_UF = 8


def kernel(state, action, weights):
    B = state.shape[0]
    F = state.shape[1] * state.shape[2]
    V = weights.shape[0]
    assert B % (_NW * _CB) == 0 and F % _UF == 0
    n_chunks = B // (_NW * _CB)

    state2 = state.reshape(_NW, n_chunks, _CB, F)
    action2 = action.astype(jnp.int32).reshape(_NW, n_chunks, _CB)
    table = weights.reshape(V, F)

    mesh = plsc.VectorSubcoreMesh(core_axis_name="c", subcore_axis_name="s")

    @functools.partial(
        pl.kernel,
        mesh=mesh,
        compiler_params=pltpu.CompilerParams(
            use_tc_tiling_on_sc=False, needs_layout_passes=False),
        out_type=jax.ShapeDtypeStruct((_NW, n_chunks, _CB), jnp.float32),
        scratch_types=[
            pltpu.VMEM((n_chunks, _CB), jnp.int32),
            pltpu.VMEM((_CB, F), jnp.float32),
            pltpu.VMEM((_CB, F), jnp.float32),
            pltpu.VMEM((_CB, F), jnp.float32),
            pltpu.VMEM((_CB, F), jnp.float32),
            pltpu.VMEM((n_chunks, _CB), jnp.float32),
            pltpu.SemaphoreType.DMA,
            pltpu.SemaphoreType.DMA,
            pltpu.SemaphoreType.DMA,
            pltpu.SemaphoreType.DMA,
        ],
    )
    def qtable(state_hbm, action_hbm, table_hbm, out_hbm,
               idx_v, w0, s0, w1, s1, obuf, sw0, ss0, sw1, ss1):
        wid = lax.axis_index("s") * _NC + lax.axis_index("c")
        pltpu.sync_copy(action_hbm.at[wid], idx_v)
        bufs = ((w0, s0, sw0, ss0), (w1, s1, sw1, ss1))

        pending = {}

        def start(c):
            wb, sb, semw, sems = bufs[c % 2]
            hw = pltpu.make_async_copy(table_hbm.at[idx_v.at[c]], wb, semw)
            hw.start()
            hs = pltpu.make_async_copy(state_hbm.at[wid, c], sb, sems)
            hs.start()
            pending[c] = (hw, hs)

        lane = lax.broadcasted_iota(jnp.int32, (16,), 0)
        zf = jnp.zeros((16,), jnp.float32)
        zi = jnp.zeros((16,), jnp.int32)

        start(0)
        if n_chunks > 1:
            start(1)
        for c in range(n_chunks):
            hw, hs = pending.pop(c)
            hw.wait()
            hs.wait()
            wb, sb, _, _ = bufs[c % 2]
            for g in range(_CB // 16):
                rows = lane + (g * 16)

                def fbody(i, acc, rows=rows, wb=wb, sb=sb):
                    f0 = i * _UF
                    for u in range(_UF):
                        col = zi + (f0 + u)
                        w = plsc.load_gather(wb, [rows, col])
                        s = plsc.load_gather(sb, [rows, col])
                        acc = acc + w * s
                    return acc

                acc = lax.fori_loop(0, F // _UF, fbody, zf)
                obuf[c, pl.ds(g * 16, 16)] = acc
            if c + 2 < n_chunks:
                start(c + 2)
        pltpu.sync_copy(obuf, out_hbm.at[wid])

    out = qtable(state2, action2, table)
    return out.reshape(B)


# speedup vs baseline: 6.3074x; 1.5422x over previous
import functools

import jax
import jax.numpy as jnp
from jax import lax
from jax.experimental import pallas as pl
from jax.experimental.pallas import tpu as pltpu
from jax.experimental.pallas import tpu_sc as plsc

_NC = 2
_NS = 16
_NW = _NC * _NS
_CBS = 128
_CBW = 64
_UF = 8


def kernel(state, action, weights):
    B, F1, F2 = state.shape
    F = F1 * F2
    V = weights.shape[0]
    assert B % (_NW * _CBS) == 0 and F % 128 == 0 and F % _UF == 0
    ns_chunks = B // (_NW * _CBS)
    nw_per_s = _CBS // _CBW
    nw_chunks = ns_chunks * nw_per_s
    b_per_w = ns_chunks * _CBS

    stateT = state.transpose(1, 2, 0).reshape(F, B)
    action32 = action.astype(jnp.int32)
    table = weights.reshape(V, F).reshape(V, F // 128, 128)

    mesh = plsc.VectorSubcoreMesh(core_axis_name="c", subcore_axis_name="s")

    @functools.partial(
        pl.kernel,
        mesh=mesh,
        compiler_params=pltpu.CompilerParams(needs_layout_passes=False),
        out_type=jax.ShapeDtypeStruct((B,), jnp.float32),
        scratch_types=[
            pltpu.VMEM((b_per_w,), jnp.int32),
            pltpu.VMEM((_CBW, F // 128, 128), jnp.float32),
            pltpu.VMEM((_CBW, F // 128, 128), jnp.float32),
            pltpu.VMEM((F, _CBS), jnp.float32),
            pltpu.VMEM((F, _CBS), jnp.float32),
            pltpu.VMEM((b_per_w,), jnp.float32),
            pltpu.SemaphoreType.DMA,
            pltpu.SemaphoreType.DMA,
            pltpu.SemaphoreType.DMA,
            pltpu.SemaphoreType.DMA,
        ],
    )
    def qtable(state_hbm, action_hbm, table_hbm, out_hbm,
               idx_v, w0, w1, s0, s1, obuf, sw0, sw1, ss0, ss1):
        wid = lax.axis_index("s") * _NC + lax.axis_index("c")
        base = wid * b_per_w
        pltpu.sync_copy(action_hbm.at[pl.ds(base, b_per_w)], idx_v)
        wbufs = ((w0, sw0), (w1, sw1))
        sbufs = ((s0, ss0), (s1, ss1))

        pending_w, pending_s = {}, {}

        def start_w(cw):
            wb, sem = wbufs[cw % 2]
            h = pltpu.make_async_copy(
                table_hbm.at[idx_v.at[pl.ds(cw * _CBW, _CBW)]], wb, sem)
            h.start()
            pending_w[cw] = h

        def start_s(cs):
            sb, sem = sbufs[cs % 2]
            h = pltpu.make_async_copy(
                state_hbm.at[:, pl.ds(base + cs * _CBS, _CBS)], sb, sem)
            h.start()
            pending_s[cs] = h

        lane = lax.broadcasted_iota(jnp.int32, (16,), 0)
        zf = jnp.zeros((16,), jnp.float32)
        zi = jnp.zeros((16,), jnp.int32)

        start_s(0)
        if ns_chunks > 1:
            start_s(1)
        start_w(0)
        if nw_chunks > 1:
            start_w(1)

        for cs in range(ns_chunks):
            pending_s.pop(cs).wait()
            sb = sbufs[cs % 2][0]
            for h in range(nw_per_s):
                cw = cs * nw_per_s + h
                pending_w.pop(cw).wait()
                wb = wbufs[cw % 2][0]
                for g in range(_CBW // 16):
                    rows = lane + (g * 16)
                    col0 = h * _CBW + g * 16

                    def fbody(i, acc, rows=rows, wb=wb, sb=sb, col0=col0):
                        f0 = i * _UF
                        for u in range(_UF):
                            f = f0 + u
                            c1 = zi + lax.shift_right_logical(f, 7)
                            c2 = zi + lax.bitwise_and(f, 127)
                            w = plsc.load_gather(wb, [rows, c1, c2])
                            s = sb[f, pl.ds(col0, 16)]
                            acc = acc + w * s
                        return acc

                    acc = lax.fori_loop(0, F // _UF, fbody, zf)
                    obuf[pl.ds(cw * _CBW + g * 16, 16)] = acc
                if cw + 2 < nw_chunks:
                    start_w(cw + 2)
            if cs + 2 < ns_chunks:
                start_s(cs + 2)
        pltpu.sync_copy(obuf, out_hbm.at[pl.ds(base, b_per_w)])

    return qtable(stateT, action32, table)
